# Initial kernel scaffold; baseline (speedup 1.0000x reference)
import jax
import jax.numpy as jnp
from jax.experimental import pallas as pl


def kernel(input, his_enc, loc, scale, w_p, w_l, w_s, w_u, s_sign, W1, b1, W2, b2):
    raise NotImplementedError("write your pallas kernel here")


# speedup vs baseline: 4.9920x; 4.9920x over previous
import functools

import jax
import jax.numpy as jnp
from jax.experimental import pallas as pl
from jax.experimental.pallas import tpu as pltpu

_F32 = jnp.float32


def _prologue_kernel(g, w_p_ref, w_l_ref, w_s_ref, w_u_ref, s_sign_ref,
                     scale_ref, loc_ref,
                     w_eff_ref, b_eff_ref, logdet0_ref):
    C = w_p_ref.shape[0]
    dt = w_p_ref.dtype
    row = jax.lax.broadcasted_iota(jnp.int32, (C, C), 0)
    col = jax.lax.broadcasted_iota(jnp.int32, (C, C), 1)
    eye = (row == col)
    w_s = w_s_ref[...]
    s_sign = s_sign_ref[...]
    d = s_sign * jnp.exp(w_s)
    L = jnp.where(row > col, w_l_ref[...], jnp.where(eye, 1.0, 0.0).astype(dt))
    U = jnp.where(row < col, w_u_ref[...],
                  jnp.where(eye, jnp.broadcast_to(d, (C, C)), 0.0).astype(dt))
    weight = jnp.dot(jnp.dot(w_p_ref[...], L, preferred_element_type=_F32),
                     U, preferred_element_type=_F32)
    scale = scale_ref[...]
    loc = loc_ref[...]
    w_eff_ref[...] = weight * scale
    b_eff_ref[...] = jnp.dot(weight, (scale * loc).reshape(C, 1).astype(dt),
                             preferred_element_type=_F32)
    ld0 = g * (jnp.sum(jnp.log(jnp.abs(scale))) + jnp.sum(w_s))
    logdet0_ref[...] = ld0.reshape(1, 1)


def _main_kernel(in_ref, his_ref, w_eff_ref, b_eff_ref, w1_ref, b1_ref,
                 w2_ref, b2_ref, logdet0_ref,
                 out_ref, ld_ref):
    C = in_ref.shape[1]
    Ch = C // 2
    x = in_ref[0]
    he = his_ref[0]
    y = jnp.dot(w_eff_ref[...], x, preferred_element_type=_F32) + b_eff_ref[...]
    in_a = y[:Ch]
    in_b = y[Ch:]
    x2 = jnp.concatenate([he, in_b], axis=0)
    h1 = jnp.dot(w1_ref[...], x2, preferred_element_type=_F32) + b1_ref[...]
    h1 = 0.5 * h1 * (1.0 + jax.lax.erf(h1 * 0.7071067811865476))
    h2 = jnp.dot(w2_ref[...], h1, preferred_element_type=_F32) + b2_ref[...]
    h2 = jax.nn.sigmoid(h2)
    log_s = h2[:Ch]
    t = h2[Ch:]
    out_ref[0, :Ch] = jnp.exp(log_s) * in_a + t
    out_ref[0, Ch:] = in_b
    ld_ref[...] = (logdet0_ref[0, 0] + jnp.sum(log_s)).reshape(1, 1, 1)


@functools.partial(jax.jit, static_argnames=("interpret",))
def _flow(input, his_enc, loc, scale, w_p, w_l, w_s, w_u, s_sign, W1, b1, W2,
          b2, interpret=False):
    B, C, G = input.shape
    H = W1.shape[0]
    dt = input.dtype

    w_eff, b_eff, logdet0 = pl.pallas_call(
        functools.partial(_prologue_kernel, float(G)),
        out_shape=(
            jax.ShapeDtypeStruct((C, C), dt),
            jax.ShapeDtypeStruct((C, 1), dt),
            jax.ShapeDtypeStruct((1, 1), dt),
        ),
        interpret=interpret,
    )(w_p, w_l, w_s.reshape(1, C), w_u, s_sign.reshape(1, C),
      scale.reshape(1, C), loc.reshape(1, C))

    out, ld = pl.pallas_call(
        _main_kernel,
        grid=(B,),
        in_specs=[
            pl.BlockSpec((1, C, G), lambda b: (b, 0, 0)),
            pl.BlockSpec((1, C, G), lambda b: (b, 0, 0)),
            pl.BlockSpec((C, C), lambda b: (0, 0)),
            pl.BlockSpec((C, 1), lambda b: (0, 0)),
            pl.BlockSpec((H, C + C // 2), lambda b: (0, 0)),
            pl.BlockSpec((H, 1), lambda b: (0, 0)),
            pl.BlockSpec((C, H), lambda b: (0, 0)),
            pl.BlockSpec((C, 1), lambda b: (0, 0)),
            pl.BlockSpec((1, 1), lambda b: (0, 0)),
        ],
        out_specs=(
            pl.BlockSpec((1, C, G), lambda b: (b, 0, 0)),
            pl.BlockSpec((1, 1, 1), lambda b: (b, 0, 0)),
        ),
        out_shape=(
            jax.ShapeDtypeStruct((B, C, G), dt),
            jax.ShapeDtypeStruct((B, 1, 1), dt),
        ),
        compiler_params=pltpu.CompilerParams(
            dimension_semantics=("parallel",),
        ),
        interpret=interpret,
    )(input, his_enc, w_eff, b_eff, W1, b1.reshape(H, 1), W2,
      b2.reshape(C, 1), logdet0)

    return out, ld.reshape(B)


def kernel(input, his_enc, loc, scale, w_p, w_l, w_s, w_u, s_sign, W1, b1, W2, b2):
    return _flow(input, his_enc, loc, scale, w_p, w_l, w_s, w_u, s_sign,
                 W1, b1, W2, b2)


# speedup vs baseline: 6.5730x; 1.3167x over previous
import functools

import jax
import jax.numpy as jnp
from jax.experimental import pallas as pl
from jax.experimental.pallas import tpu as pltpu

_F32 = jnp.float32


def _prologue_kernel(g, w_p_ref, w_l_ref, w_s_ref, w_u_ref, s_sign_ref,
                     scale_ref, loc_ref,
                     w_eff_ref, b_eff_ref, logdet0_ref):
    C = w_p_ref.shape[0]
    dt = w_p_ref.dtype
    row = jax.lax.broadcasted_iota(jnp.int32, (C, C), 0)
    col = jax.lax.broadcasted_iota(jnp.int32, (C, C), 1)
    eye = (row == col)
    w_s = w_s_ref[...]
    s_sign = s_sign_ref[...]
    d = s_sign * jnp.exp(w_s)
    L = jnp.where(row > col, w_l_ref[...], jnp.where(eye, 1.0, 0.0).astype(dt))
    U = jnp.where(row < col, w_u_ref[...],
                  jnp.where(eye, jnp.broadcast_to(d, (C, C)), 0.0).astype(dt))
    weight = jnp.dot(jnp.dot(w_p_ref[...], L, preferred_element_type=_F32),
                     U, preferred_element_type=_F32)
    scale = scale_ref[...]
    loc = loc_ref[...]
    w_eff_ref[...] = weight * scale
    b_eff_ref[...] = jnp.dot(weight, (scale * loc).reshape(C, 1).astype(dt),
                             preferred_element_type=_F32)
    ld0 = g * (jnp.sum(jnp.log(jnp.abs(scale))) + jnp.sum(w_s))
    logdet0_ref[...] = ld0.reshape(1, 1)


def _main_kernel(in_ref, his_ref, w_eff_ref, b_eff_ref, w1_ref, b1_ref,
                 w2_ref, b2_ref, logdet0_ref,
                 out_ref, ld_ref):
    BB, C, _ = in_ref.shape
    Ch = C // 2
    w_eff = w_eff_ref[...]
    b_eff = b_eff_ref[...]
    w1 = w1_ref[...]
    b1 = b1_ref[...]
    w2 = w2_ref[...]
    b2 = b2_ref[...]
    ld0 = logdet0_ref[0, 0]
    for i in range(BB):
        x = in_ref[i]
        he = his_ref[i]
        y = jnp.dot(w_eff, x, preferred_element_type=_F32) + b_eff
        in_a = y[:Ch]
        in_b = y[Ch:]
        x2 = jnp.concatenate([he, in_b], axis=0)
        h1 = jnp.dot(w1, x2, preferred_element_type=_F32) + b1
        h1 = 0.5 * h1 * (1.0 + jax.lax.erf(h1 * 0.7071067811865476))
        h2 = jnp.dot(w2, h1, preferred_element_type=_F32) + b2
        h2 = jax.nn.sigmoid(h2)
        log_s = h2[:Ch]
        t = h2[Ch:]
        out_ref[i, :Ch] = jnp.exp(log_s) * in_a + t
        out_ref[i, Ch:] = in_b
        ld_ref[i:i + 1] = (ld0 + jnp.sum(log_s)).reshape(1, 1, 1)


@functools.partial(jax.jit, static_argnames=("interpret",))
def _flow(input, his_enc, loc, scale, w_p, w_l, w_s, w_u, s_sign, W1, b1, W2,
          b2, interpret=False):
    B, C, G = input.shape
    H = W1.shape[0]
    dt = input.dtype

    w_eff, b_eff, logdet0 = pl.pallas_call(
        functools.partial(_prologue_kernel, float(G)),
        out_shape=(
            jax.ShapeDtypeStruct((C, C), dt),
            jax.ShapeDtypeStruct((C, 1), dt),
            jax.ShapeDtypeStruct((1, 1), dt),
        ),
        interpret=interpret,
    )(w_p, w_l, w_s.reshape(1, C), w_u, s_sign.reshape(1, C),
      scale.reshape(1, C), loc.reshape(1, C))

    BB = 4
    out, ld = pl.pallas_call(
        _main_kernel,
        grid=(B // BB,),
        in_specs=[
            pl.BlockSpec((BB, C, G), lambda b: (b, 0, 0)),
            pl.BlockSpec((BB, C, G), lambda b: (b, 0, 0)),
            pl.BlockSpec((C, C), lambda b: (0, 0)),
            pl.BlockSpec((C, 1), lambda b: (0, 0)),
            pl.BlockSpec((H, C + C // 2), lambda b: (0, 0)),
            pl.BlockSpec((H, 1), lambda b: (0, 0)),
            pl.BlockSpec((C, H), lambda b: (0, 0)),
            pl.BlockSpec((C, 1), lambda b: (0, 0)),
            pl.BlockSpec((1, 1), lambda b: (0, 0)),
        ],
        out_specs=(
            pl.BlockSpec((BB, C, G), lambda b: (b, 0, 0)),
            pl.BlockSpec((BB, 1, 1), lambda b: (b, 0, 0)),
        ),
        out_shape=(
            jax.ShapeDtypeStruct((B, C, G), dt),
            jax.ShapeDtypeStruct((B, 1, 1), dt),
        ),
        compiler_params=pltpu.CompilerParams(
            dimension_semantics=("parallel",),
        ),
        interpret=interpret,
    )(input, his_enc, w_eff, b_eff, W1, b1.reshape(H, 1), W2,
      b2.reshape(C, 1), logdet0)

    return out, ld.reshape(B)


def kernel(input, his_enc, loc, scale, w_p, w_l, w_s, w_u, s_sign, W1, b1, W2, b2):
    return _flow(input, his_enc, loc, scale, w_p, w_l, w_s, w_u, s_sign,
                 W1, b1, W2, b2)


# speedup vs baseline: 6.6707x; 1.0149x over previous
import functools

import jax
import jax.numpy as jnp
from jax.experimental import pallas as pl
from jax.experimental.pallas import tpu as pltpu

_F32 = jnp.float32


def _prologue_kernel(g, w_p_ref, w_l_ref, w_s_ref, w_u_ref, s_sign_ref,
                     scale_ref, loc_ref,
                     w_eff_ref, b_eff_ref, logdet0_ref):
    C = w_p_ref.shape[0]
    dt = w_p_ref.dtype
    row = jax.lax.broadcasted_iota(jnp.int32, (C, C), 0)
    col = jax.lax.broadcasted_iota(jnp.int32, (C, C), 1)
    eye = (row == col)
    w_s = w_s_ref[...]
    s_sign = s_sign_ref[...]
    d = s_sign * jnp.exp(w_s)
    L = jnp.where(row > col, w_l_ref[...], jnp.where(eye, 1.0, 0.0).astype(dt))
    U = jnp.where(row < col, w_u_ref[...],
                  jnp.where(eye, jnp.broadcast_to(d, (C, C)), 0.0).astype(dt))
    weight = jnp.dot(jnp.dot(w_p_ref[...], L, preferred_element_type=_F32),
                     U, preferred_element_type=_F32)
    scale = scale_ref[...]
    loc = loc_ref[...]
    w_eff_ref[...] = weight * scale
    b_eff_ref[...] = jnp.dot(weight, (scale * loc).reshape(C, 1).astype(dt),
                             preferred_element_type=_F32)
    ld0 = g * (jnp.sum(jnp.log(jnp.abs(scale))) + jnp.sum(w_s))
    logdet0_ref[...] = ld0.reshape(1, 1)


def _main_kernel(in_ref, his_ref, w_eff_ref, b_eff_ref, w1_ref, b1_ref,
                 w2_ref, b2_ref, logdet0_ref,
                 out_ref, ld_ref):
    BB, C, _ = in_ref.shape
    Ch = C // 2
    w_eff = w_eff_ref[...]
    b_eff = b_eff_ref[...]
    w1 = w1_ref[...]
    b1 = b1_ref[...]
    w2 = w2_ref[...]
    b2 = b2_ref[...]
    ld0 = logdet0_ref[0, 0]
    for i in range(BB):
        x = in_ref[i]
        he = his_ref[i]
        y = jnp.dot(w_eff, x, preferred_element_type=_F32) + b_eff
        in_a = y[:Ch]
        in_b = y[Ch:]
        x2 = jnp.concatenate([he, in_b], axis=0)
        h1 = jnp.dot(w1, x2, preferred_element_type=_F32) + b1
        h1 = 0.5 * h1 * (1.0 + jax.lax.erf(h1 * 0.7071067811865476))
        h2 = jnp.dot(w2, h1, preferred_element_type=_F32) + b2
        h2 = jax.nn.sigmoid(h2)
        log_s = h2[:Ch]
        t = h2[Ch:]
        out_ref[i, :Ch] = jnp.exp(log_s) * in_a + t
        out_ref[i, Ch:] = in_b
        ld_ref[i:i + 1] = (ld0 + jnp.sum(log_s)).reshape(1, 1, 1)


@functools.partial(jax.jit, static_argnames=("interpret",))
def _flow(input, his_enc, loc, scale, w_p, w_l, w_s, w_u, s_sign, W1, b1, W2,
          b2, interpret=False):
    B, C, G = input.shape
    H = W1.shape[0]
    dt = input.dtype

    w_eff, b_eff, logdet0 = pl.pallas_call(
        functools.partial(_prologue_kernel, float(G)),
        out_shape=(
            jax.ShapeDtypeStruct((C, C), dt),
            jax.ShapeDtypeStruct((C, 1), dt),
            jax.ShapeDtypeStruct((1, 1), dt),
        ),
        interpret=interpret,
    )(w_p, w_l, w_s.reshape(1, C), w_u, s_sign.reshape(1, C),
      scale.reshape(1, C), loc.reshape(1, C))

    BB = 8
    out, ld = pl.pallas_call(
        _main_kernel,
        grid=(B // BB,),
        in_specs=[
            pl.BlockSpec((BB, C, G), lambda b: (b, 0, 0)),
            pl.BlockSpec((BB, C, G), lambda b: (b, 0, 0)),
            pl.BlockSpec((C, C), lambda b: (0, 0)),
            pl.BlockSpec((C, 1), lambda b: (0, 0)),
            pl.BlockSpec((H, C + C // 2), lambda b: (0, 0)),
            pl.BlockSpec((H, 1), lambda b: (0, 0)),
            pl.BlockSpec((C, H), lambda b: (0, 0)),
            pl.BlockSpec((C, 1), lambda b: (0, 0)),
            pl.BlockSpec((1, 1), lambda b: (0, 0)),
        ],
        out_specs=(
            pl.BlockSpec((BB, C, G), lambda b: (b, 0, 0)),
            pl.BlockSpec((BB, 1, 1), lambda b: (b, 0, 0)),
        ),
        out_shape=(
            jax.ShapeDtypeStruct((B, C, G), dt),
            jax.ShapeDtypeStruct((B, 1, 1), dt),
        ),
        compiler_params=pltpu.CompilerParams(
            dimension_semantics=("parallel",),
        ),
        interpret=interpret,
    )(input, his_enc, w_eff, b_eff, W1, b1.reshape(H, 1), W2,
      b2.reshape(C, 1), logdet0)

    return out, ld.reshape(B)


def kernel(input, his_enc, loc, scale, w_p, w_l, w_s, w_u, s_sign, W1, b1, W2, b2):
    return _flow(input, his_enc, loc, scale, w_p, w_l, w_s, w_u, s_sign,
                 W1, b1, W2, b2)
